# Initial kernel scaffold; baseline (speedup 1.0000x reference)
import jax
import jax.numpy as jnp
from jax.experimental import pallas as pl


def kernel(indices, embedding_matrix):
    raise NotImplementedError("write your pallas kernel here")


# speedup vs baseline: 1.1016x; 1.1016x over previous
import functools

import jax
import jax.numpy as jnp
from jax import lax
from jax.experimental import pallas as pl
from jax.experimental.pallas import tpu as pltpu
from jax.experimental.pallas import tpu_sc as plsc

_VOCAB = 1000000
_EMBED = 32
_BATCH = 16384
_HIST = 50
_TOTAL = _BATCH * _HIST

_NC = 2
_NS = 16
_NW = _NC * _NS
_B_PER_W = _TOTAL // _NW
_CH = 1600
_NCHUNK = _B_PER_W // _CH

_mesh = plsc.VectorSubcoreMesh(core_axis_name="c", subcore_axis_name="s")


@functools.partial(
    pl.kernel,
    mesh=_mesh,
    out_type=jax.ShapeDtypeStruct((_TOTAL, _EMBED), jnp.float32),
    scratch_types=[
        pltpu.VMEM((_CH,), jnp.int32),
        pltpu.VMEM((_CH, _EMBED), jnp.float32),
        pltpu.SemaphoreType.DMA,
    ],
    compiler_params=pltpu.CompilerParams(use_tc_tiling_on_sc=False),
)
def _gather_kernel(idx_hbm, table_hbm, out_hbm, idx_v, rows_v, sem):
    wid = lax.axis_index("s") * _NC + lax.axis_index("c")
    base = wid * _B_PER_W

    def body(i, carry):
        off = base + i * _CH
        pltpu.sync_copy(idx_hbm.at[pl.ds(off, _CH)], idx_v)
        pltpu.async_copy(table_hbm.at[idx_v], rows_v, sem).wait()
        pltpu.sync_copy(rows_v, out_hbm.at[pl.ds(off, _CH)])
        return carry

    lax.fori_loop(0, _NCHUNK, body, 0)


def kernel(indices, embedding_matrix):
    flat = indices.reshape(-1).astype(jnp.int32)
    out = _gather_kernel(flat, embedding_matrix)
    return out.reshape(_BATCH, _HIST, _EMBED)


# speedup vs baseline: 1.1121x; 1.0095x over previous
import functools

import jax
import jax.numpy as jnp
from jax import lax
from jax.experimental import pallas as pl
from jax.experimental.pallas import tpu as pltpu
from jax.experimental.pallas import tpu_sc as plsc

_VOCAB = 1000000
_EMBED = 32
_BATCH = 16384
_HIST = 50
_TOTAL = _BATCH * _HIST

_NC = 2
_NS = 16
_NW = _NC * _NS
_B_PER_W = _TOTAL // _NW
_CH = 1600
_NCHUNK = _B_PER_W // _CH

_mesh = plsc.VectorSubcoreMesh(core_axis_name="c", subcore_axis_name="s")


_NBUF = 2


@functools.partial(
    pl.kernel,
    mesh=_mesh,
    out_type=jax.ShapeDtypeStruct((_TOTAL, _EMBED), jnp.float32),
    scratch_types=[
        pltpu.VMEM((_B_PER_W,), jnp.int32),
        pltpu.VMEM((_CH, _EMBED), jnp.float32),
        pltpu.VMEM((_CH, _EMBED), jnp.float32),
        pltpu.SemaphoreType.DMA,
        pltpu.SemaphoreType.DMA,
        pltpu.SemaphoreType.DMA,
        pltpu.SemaphoreType.DMA,
    ],
    compiler_params=pltpu.CompilerParams(use_tc_tiling_on_sc=False),
)
def _gather_kernel(idx_hbm, table_hbm, out_hbm, idx_v, r0, r1, g0, g1, o0, o1):
    wid = lax.axis_index("s") * _NC + lax.axis_index("c")
    base = wid * _B_PER_W

    bufs = (r0, r1)
    gsems = (g0, g1)
    osems = (o0, o1)

    pltpu.sync_copy(idx_hbm.at[pl.ds(base, _B_PER_W)], idx_v)

    gathers = [None] * _NCHUNK
    outs = [None] * _NCHUNK
    gathers[0] = pltpu.async_copy(
        table_hbm.at[idx_v.at[pl.ds(0, _CH)]], bufs[0], gsems[0]
    )
    for i in range(_NCHUNK):
        b = i % _NBUF
        if i + 1 < _NCHUNK:
            nb = (i + 1) % _NBUF
            if i + 1 >= _NBUF:
                outs[i + 1 - _NBUF].wait()
            gathers[i + 1] = pltpu.async_copy(
                table_hbm.at[idx_v.at[pl.ds((i + 1) * _CH, _CH)]],
                bufs[nb],
                gsems[nb],
            )
        gathers[i].wait()
        outs[i] = pltpu.async_copy(
            bufs[b], out_hbm.at[pl.ds(base + i * _CH, _CH)], osems[b]
        )
    for i in range(_NCHUNK - _NBUF, _NCHUNK):
        outs[i].wait()


def kernel(indices, embedding_matrix):
    flat = indices.reshape(-1).astype(jnp.int32)
    out = _gather_kernel(flat, embedding_matrix)
    return out.reshape(_BATCH, _HIST, _EMBED)


# speedup vs baseline: 1.6009x; 1.4396x over previous
import functools

import jax
import jax.numpy as jnp
from jax import lax
from jax.experimental import pallas as pl
from jax.experimental.pallas import tpu as pltpu
from jax.experimental.pallas import tpu_sc as plsc

_VOCAB = 1000000
_EMBED = 32
_BATCH = 16384
_HIST = 50

_NC = 2
_NS = 16
_NW = _NC * _NS

_G = 4
_CHUNK = _G * 128
_NBTG = _BATCH // _CHUNK
_NUNIT = _HIST * _NBTG
_U_PER_W = _NUNIT // _NW

_mesh = plsc.VectorSubcoreMesh(core_axis_name="c", subcore_axis_name="s")


@functools.partial(
    pl.kernel,
    mesh=_mesh,
    out_type=jax.ShapeDtypeStruct(
        (_HIST, _EMBED // 8, _BATCH // 128, 8, 128), jnp.float32
    ),
    scratch_types=[
        pltpu.VMEM((_CHUNK,), jnp.int32),
        pltpu.VMEM((_CHUNK,), jnp.int32),
        pltpu.VMEM((_CHUNK, _EMBED), jnp.float32),
        pltpu.VMEM((_CHUNK, _EMBED), jnp.float32),
        pltpu.VMEM((_G, _EMBED, 128), jnp.float32),
        pltpu.VMEM((_G, _EMBED, 128), jnp.float32),
        pltpu.SemaphoreType.DMA,
        pltpu.SemaphoreType.DMA,
        pltpu.SemaphoreType.DMA,
        pltpu.SemaphoreType.DMA,
    ],
    compiler_params=pltpu.CompilerParams(
        use_tc_tiling_on_sc=False, needs_layout_passes=False
    ),
)
def _gather_kernel(idx_hbm, table_hbm, out_hbm, i0, i1, r0, r1, t0, t1,
                   g0, g1, os0, os1):
    wid = lax.axis_index("s") * _NC + lax.axis_index("c")
    u0 = wid * _U_PER_W
    lanes = lax.iota(jnp.int32, 16)

    def start_gather(k, ibuf, rbuf, gsem):
        u = u0 + k
        h = u // _NBTG
        btg = u % _NBTG
        pltpu.sync_copy(idx_hbm.at[h].at[pl.ds(btg * _CHUNK, _CHUNK)], ibuf)
        return pltpu.async_copy(table_hbm.at[ibuf], rbuf, gsem)

    def wait_gather(rbuf, gsem):
        pltpu.make_async_copy(table_hbm.at[i0], rbuf, gsem).wait()

    def transpose(rows, trans):
        def tr_body(g, carry):
            bt = g // 8
            v = g % 8
            row_ids = bt * 128 + v * 16 + lanes
            for c in range(_EMBED):
                col = jnp.full((16,), c, jnp.int32)
                trans[bt, c, pl.ds(v * 16, 16)] = plsc.load_gather(
                    rows, [row_ids, col]
                )
            return carry

        lax.fori_loop(0, _G * 8, tr_body, 0)

    def fire_outs(k, trans, osem):
        u = u0 + k
        h = u // _NBTG
        btg = u % _NBTG
        d = None
        for ct in range(_EMBED // 8):
            d = pltpu.async_copy(
                trans.at[:, pl.ds(ct * 8, 8), :],
                out_hbm.at[h, ct].at[pl.ds(btg * _G, _G)],
                osem,
            )
        return d

    def drain_outs(trans, osem):
        d = pltpu.make_async_copy(
            trans.at[:, pl.ds(0, 8), :], out_hbm.at[0, 0].at[pl.ds(0, _G)],
            osem,
        )
        for _ in range(_EMBED // 8):
            d.wait()

    start_gather(0, i0, r0, g0)
    start_gather(1, i1, r1, g1)
    wait_gather(r0, g0)
    transpose(r0, t0)
    fire_outs(0, t0, os0)
    start_gather(2, i0, r0, g0)
    wait_gather(r1, g1)
    transpose(r1, t1)
    fire_outs(1, t1, os1)
    start_gather(3, i1, r1, g1)

    def pair_body(j, carry):
        drain_outs(t0, os0)
        wait_gather(r0, g0)
        transpose(r0, t0)
        fire_outs(2 * j, t0, os0)
        start_gather(2 * j + 2, i0, r0, g0)
        drain_outs(t1, os1)
        wait_gather(r1, g1)
        transpose(r1, t1)
        fire_outs(2 * j + 1, t1, os1)
        start_gather(2 * j + 3, i1, r1, g1)
        return carry

    lax.fori_loop(1, _U_PER_W // 2 - 1, pair_body, 0)

    drain_outs(t0, os0)
    wait_gather(r0, g0)
    transpose(r0, t0)
    fire_outs(_U_PER_W - 2, t0, os0)
    drain_outs(t1, os1)
    wait_gather(r1, g1)
    transpose(r1, t1)
    fire_outs(_U_PER_W - 1, t1, os1)
    drain_outs(t0, os0)
    drain_outs(t1, os1)


def kernel(indices, embedding_matrix):
    idx_t = indices.T
    out5 = _gather_kernel(idx_t, embedding_matrix)
    return out5.transpose(2, 4, 0, 1, 3).reshape(_BATCH, _HIST, _EMBED)


# speedup vs baseline: 2.4615x; 1.5376x over previous
import functools

import jax
import jax.numpy as jnp
from jax import lax
from jax.experimental import pallas as pl
from jax.experimental.pallas import tpu as pltpu
from jax.experimental.pallas import tpu_sc as plsc

_VOCAB = 1000000
_EMBED = 32
_BATCH = 16384
_HIST = 50

_NC = 2
_NS = 16
_NW = _NC * _NS

_G = 4
_CHUNK = _G * 128
_NBTG = _BATCH // _CHUNK
_NUNIT = _HIST * _NBTG
_U_PER_W = _NUNIT // _NW

_mesh = plsc.VectorSubcoreMesh(core_axis_name="c", subcore_axis_name="s")


@functools.partial(
    pl.kernel,
    mesh=_mesh,
    out_type=jax.ShapeDtypeStruct(
        (_HIST, _EMBED // 8, _BATCH // 128, 8, 128), jnp.float32
    ),
    scratch_types=[
        pltpu.VMEM((_CHUNK,), jnp.int32),
        pltpu.VMEM((_CHUNK,), jnp.int32),
        pltpu.VMEM((_CHUNK, _EMBED), jnp.float32),
        pltpu.VMEM((_CHUNK, _EMBED), jnp.float32),
        pltpu.VMEM((_G, _EMBED, 129), jnp.float32),
        pltpu.VMEM((_G, _EMBED, 129), jnp.float32),
        pltpu.SemaphoreType.DMA,
        pltpu.SemaphoreType.DMA,
        pltpu.SemaphoreType.DMA,
        pltpu.SemaphoreType.DMA,
    ],
    compiler_params=pltpu.CompilerParams(
        use_tc_tiling_on_sc=False, needs_layout_passes=False
    ),
)
def _gather_kernel(idx_hbm, table_hbm, out_hbm, i0, i1, r0, r1, t0, t1,
                   g0, g1, os0, os1):
    wid = lax.axis_index("s") * _NC + lax.axis_index("c")
    u0 = wid * _U_PER_W
    lanes = lax.iota(jnp.int32, 16)

    def start_gather(k, ibuf, rbuf, gsem):
        u = u0 + k
        h = u // _NBTG
        btg = u % _NBTG
        pltpu.sync_copy(idx_hbm.at[h].at[pl.ds(btg * _CHUNK, _CHUNK)], ibuf)
        return pltpu.async_copy(table_hbm.at[ibuf], rbuf, gsem)

    def wait_gather(rbuf, gsem):
        pltpu.make_async_copy(table_hbm.at[i0], rbuf, gsem).wait()

    lanes_hi = lanes + 16

    def transpose(rows, trans):
        def tr_body(g, carry):
            bt = g // 8
            v = g % 8
            bt_v = jnp.full((16,), 0, jnp.int32) + bt
            for t in range(16):
                bi = v * 16 + t
                j = bt * 128 + bi
                bi_v = jnp.full((16,), 0, jnp.int32) + bi
                plsc.store_scatter(
                    trans, [bt_v, lanes, bi_v], rows[j, pl.ds(0, 16)]
                )
                plsc.store_scatter(
                    trans, [bt_v, lanes_hi, bi_v], rows[j, pl.ds(16, 16)]
                )
            return carry

        lax.fori_loop(0, _G * 8, tr_body, 0)

    def fire_outs(k, trans, osem):
        u = u0 + k
        h = u // _NBTG
        btg = u % _NBTG
        d = None
        for ct in range(_EMBED // 8):
            d = pltpu.async_copy(
                trans.at[:, pl.ds(ct * 8, 8), pl.ds(0, 128)],
                out_hbm.at[h, ct].at[pl.ds(btg * _G, _G)],
                osem,
            )
        return d

    def drain_outs(trans, osem):
        d = pltpu.make_async_copy(
            trans.at[:, pl.ds(0, 8), pl.ds(0, 128)], out_hbm.at[0, 0].at[pl.ds(0, _G)],
            osem,
        )
        for _ in range(_EMBED // 8):
            d.wait()

    start_gather(0, i0, r0, g0)
    start_gather(1, i1, r1, g1)
    wait_gather(r0, g0)
    transpose(r0, t0)
    fire_outs(0, t0, os0)
    start_gather(2, i0, r0, g0)
    wait_gather(r1, g1)
    transpose(r1, t1)
    fire_outs(1, t1, os1)
    start_gather(3, i1, r1, g1)

    def pair_body(j, carry):
        drain_outs(t0, os0)
        wait_gather(r0, g0)
        transpose(r0, t0)
        fire_outs(2 * j, t0, os0)
        start_gather(2 * j + 2, i0, r0, g0)
        drain_outs(t1, os1)
        wait_gather(r1, g1)
        transpose(r1, t1)
        fire_outs(2 * j + 1, t1, os1)
        start_gather(2 * j + 3, i1, r1, g1)
        return carry

    lax.fori_loop(1, _U_PER_W // 2 - 1, pair_body, 0)

    drain_outs(t0, os0)
    wait_gather(r0, g0)
    transpose(r0, t0)
    fire_outs(_U_PER_W - 2, t0, os0)
    drain_outs(t1, os1)
    wait_gather(r1, g1)
    transpose(r1, t1)
    fire_outs(_U_PER_W - 1, t1, os1)
    drain_outs(t0, os0)
    drain_outs(t1, os1)


def kernel(indices, embedding_matrix):
    idx_t = indices.T
    out5 = _gather_kernel(idx_t, embedding_matrix)
    return out5.transpose(2, 4, 0, 1, 3).reshape(_BATCH, _HIST, _EMBED)
